# baseline (speedup 1.0000x reference)
import functools
import jax
import jax.numpy as jnp
from jax import lax
from jax.experimental import pallas as pl
from jax.experimental.pallas import tpu as pltpu
from jax.experimental.pallas import tpu_sc as plsc

N = 100000
E = 50
L = 16
NC = 2
NS = 16
NW = NC * NS
CH = 512
NFULL = 6
BASE_ROWS = 3120
EXTRA = 10
SPAN_A = BASE_ROWS + 16
TAIL_A = SPAN_A - NFULL * CH
TAIL_B = BASE_ROWS - NFULL * CH
G = 8
TILE_ROWS = G * L
TILES = CH // TILE_ROWS


def _body(x_hbm, ae_hbm, out_hbm, xbuf0, xbuf1, aebuf, outbuf, sem0, sem1):
    wid = lax.axis_index("s") * NC + lax.axis_index("c")
    base = wid * BASE_ROWS + jnp.minimum(wid, EXTRA) * 16
    lanes = lax.iota(jnp.int32, L)
    zero = jnp.zeros((L,), jnp.float32)

    pltpu.sync_copy(ae_hbm, aebuf)

    xbufs = (xbuf0, xbuf1)
    sems = (sem0, sem1)

    def start_in(ii, b):
        pltpu.async_copy(x_hbm.at[pl.ds(base + ii * CH, CH), :], xbufs[b],
                         sems[b])

    def wait_in(b):
        pltpu.make_async_copy(x_hbm.at[pl.ds(0, CH), :], xbufs[b],
                              sems[b]).wait()

    def compute_chunk(xb, out0):
        @pl.loop(0, TILES)
        def _(t):
            tbase = t * TILE_ROWS
            rowv = [lanes + (tbase + L * j) for j in range(G)]
            accs = [zero] * G
            for k in range(E):
                aev = aebuf[pl.ds(L * k, L)]
                ck = jnp.full((L,), k, jnp.int32)
                for j in range(G):
                    v = plsc.load_gather(xb, [rowv[j], ck])
                    accs[j] = accs[j] + v * aev
            for j in range(G):
                outbuf[pl.ds(out0 + tbase + L * j, L)] = accs[j]

    def compute_rows(xb, out0, nrows):
        for g in range(nrows // L):
            rowv = lanes + g * L
            acc = zero
            for k in range(E):
                aev = aebuf[pl.ds(L * k, L)]
                ck = jnp.full((L,), k, jnp.int32)
                acc = acc + plsc.load_gather(xb, [rowv, ck]) * aev
            outbuf[pl.ds(out0 + g * L, L)] = acc

    start_in(0, 0)

    @pl.loop(0, NFULL, step=2)
    def _(i):
        for b in (0, 1):
            ii = i + b

            @pl.when(ii < NFULL - 1)
            def _():
                start_in(ii + 1, 1 - b)

            wait_in(b)
            compute_chunk(xbufs[b], ii * CH)

    tail_row = base + NFULL * CH

    @pl.when(wid < EXTRA)
    def _():
        pltpu.sync_copy(x_hbm.at[pl.ds(tail_row, TAIL_A), :],
                        xbuf0.at[pl.ds(0, TAIL_A), :])
        compute_rows(xbuf0, NFULL * CH, TAIL_A)
        pltpu.sync_copy(outbuf.at[pl.ds(0, SPAN_A)],
                        out_hbm.at[pl.ds(base, SPAN_A)])

    @pl.when(wid >= EXTRA)
    def _():
        pltpu.sync_copy(x_hbm.at[pl.ds(tail_row, TAIL_B), :],
                        xbuf0.at[pl.ds(0, TAIL_B), :])
        compute_rows(xbuf0, NFULL * CH, TAIL_B)
        pltpu.sync_copy(outbuf.at[pl.ds(0, BASE_ROWS)],
                        out_hbm.at[pl.ds(base, BASE_ROWS)])


@functools.partial(
    pl.kernel,
    out_type=jax.ShapeDtypeStruct((N,), jnp.float32),
    mesh=plsc.VectorSubcoreMesh(core_axis_name="c", subcore_axis_name="s"),
    compiler_params=pltpu.CompilerParams(
        needs_layout_passes=False, use_tc_tiling_on_sc=False),
    scratch_types=[
        pltpu.VMEM((CH, E), jnp.float32),
        pltpu.VMEM((CH, E), jnp.float32),
        pltpu.VMEM((E * L,), jnp.float32),
        pltpu.VMEM((SPAN_A,), jnp.float32),
        pltpu.SemaphoreType.DMA,
        pltpu.SemaphoreType.DMA,
    ],
)
def _sc_matvec(x, ae_exp, out_flat, xbuf0, xbuf1, aebuf, outbuf, sem0, sem1):
    _body(x, ae_exp, out_flat, xbuf0, xbuf1, aebuf, outbuf, sem0, sem1)


@jax.jit
def kernel(x, atomic_energies):
    ae_exp = jnp.broadcast_to(atomic_energies[:, None], (E, L)).reshape(E * L)
    out = _sc_matvec(x, ae_exp)
    return out[:, None]


# speedup vs baseline: 1.8567x; 1.8567x over previous
import functools
import jax
import jax.numpy as jnp
from jax import lax
from jax.experimental import pallas as pl
from jax.experimental.pallas import tpu as pltpu
from jax.experimental.pallas import tpu_sc as plsc

N = 100000
E = 50
EP = 128
L = 16
NC = 2
NS = 16
NW = NC * NS
CH = 256
G = 8
TILE_ROWS = G * L
TILES = CH // TILE_ROWS

BASE_ROWS = (N // (NW * L)) * L
EXTRA = (N - NW * BASE_ROWS) // L
SPAN_A = BASE_ROWS + L
NFULL = BASE_ROWS // CH
TAIL_A = SPAN_A - NFULL * CH
TAIL_B = BASE_ROWS - NFULL * CH


def _body(x_hbm, ae_hbm, out_hbm, xbuf0, xbuf1, aebuf, outbuf, sem0, sem1):
    wid = lax.axis_index("s") * NC + lax.axis_index("c")
    base = wid * BASE_ROWS + jnp.minimum(wid, EXTRA) * L
    lanes = lax.iota(jnp.int32, L)
    zero = jnp.zeros((L,), jnp.float32)

    pltpu.sync_copy(ae_hbm, aebuf)

    colvs = []
    for k in range(E):
        cv = lanes + k
        colvs.append(jnp.where(cv >= E, cv - E, cv))

    xbufs = (xbuf0, xbuf1)
    sems = (sem0, sem1)

    def start_in(ii, b):
        pltpu.async_copy(x_hbm.at[pl.ds(base + ii * CH, CH), :],
                         xbufs[b], sems[b])

    def wait_in(b):
        pltpu.make_async_copy(x_hbm.at[pl.ds(0, CH), :],
                              xbufs[b], sems[b]).wait()

    def compute_chunk(xb, out0):
        @pl.loop(0, TILES)
        def _(t):
            tbase = t * TILE_ROWS
            rowv = [lanes + (tbase + L * j) for j in range(G)]
            accs = [zero] * G
            for k in range(E):
                aev = aebuf[pl.ds(L * k, L)]
                for j in range(G):
                    v = plsc.load_gather(xb, [rowv[j], colvs[k]])
                    accs[j] = accs[j] + v * aev
            for j in range(G):
                outbuf[pl.ds(out0 + tbase + L * j, L)] = accs[j]

    def compute_rows(xb, out0, nrows):
        @pl.loop(0, nrows // L)
        def _(g):
            rowv = lanes + g * L
            acc0 = zero
            acc1 = zero
            for k in range(0, E, 2):
                aev0 = aebuf[pl.ds(L * k, L)]
                aev1 = aebuf[pl.ds(L * (k + 1), L)]
                acc0 = acc0 + plsc.load_gather(xb, [rowv, colvs[k]]) * aev0
                acc1 = acc1 + plsc.load_gather(xb, [rowv, colvs[k + 1]]) * aev1
            outbuf[pl.ds(out0 + g * L, L)] = acc0 + acc1

    start_in(0, 0)

    @pl.loop(0, NFULL, step=2)
    def _(i):
        for b in (0, 1):
            ii = i + b

            @pl.when(ii < NFULL - 1)
            def _():
                start_in(ii + 1, 1 - b)

            wait_in(b)
            compute_chunk(xbufs[b], ii * CH)

    tail_row = base + NFULL * CH

    @pl.when(wid < EXTRA)
    def _():
        pltpu.sync_copy(x_hbm.at[pl.ds(tail_row, TAIL_A), :],
                        xbuf0.at[pl.ds(0, TAIL_A), :])
        compute_rows(xbuf0, NFULL * CH, TAIL_A)
        pltpu.sync_copy(outbuf.at[pl.ds(0, SPAN_A)],
                        out_hbm.at[pl.ds(base, SPAN_A)])

    @pl.when(wid >= EXTRA)
    def _():
        pltpu.sync_copy(x_hbm.at[pl.ds(tail_row, TAIL_B), :],
                        xbuf0.at[pl.ds(0, TAIL_B), :])
        compute_rows(xbuf0, NFULL * CH, TAIL_B)
        pltpu.sync_copy(outbuf.at[pl.ds(0, BASE_ROWS)],
                        out_hbm.at[pl.ds(base, BASE_ROWS)])


@functools.partial(
    pl.kernel,
    out_type=jax.ShapeDtypeStruct((N,), jnp.float32),
    mesh=plsc.VectorSubcoreMesh(core_axis_name="c", subcore_axis_name="s"),
    compiler_params=pltpu.CompilerParams(
        needs_layout_passes=False, use_tc_tiling_on_sc=True),
    scratch_types=[
        pltpu.VMEM((CH, E), jnp.float32),
        pltpu.VMEM((CH, E), jnp.float32),
        pltpu.VMEM((E * L,), jnp.float32),
        pltpu.VMEM((SPAN_A,), jnp.float32),
        pltpu.SemaphoreType.DMA,
        pltpu.SemaphoreType.DMA,
    ],
)
def _sc_matvec(x, ae_rot, out_flat, xbuf0, xbuf1, aebuf, outbuf, sem0, sem1):
    _body(x, ae_rot, out_flat, xbuf0, xbuf1, aebuf, outbuf, sem0, sem1)


@jax.jit
def kernel(x, atomic_energies):
    k_idx = jnp.arange(E)[:, None]
    i_idx = jnp.arange(L)[None, :]
    ae_rot = atomic_energies[(k_idx + i_idx) % E].reshape(E * L)
    out = _sc_matvec(x, ae_rot)
    return out[:, None]


# speedup vs baseline: 4.0362x; 2.1738x over previous
import functools
import jax
import jax.numpy as jnp
from jax import lax
from jax.experimental import pallas as pl
from jax.experimental.pallas import tpu as pltpu
from jax.experimental.pallas import tpu_sc as plsc

N = 100000
E = 50
L = 16
NC = 2
NS = 16
NW = NC * NS
G = 8
TILE_ROWS = G * L
NSLAB = E // 8

TC_ROWS = 63488
BC = 2048
TC_GRID = TC_ROWS // BC

CH = 1024
SC_ROWS = N - TC_ROWS
REM = 32
EXTRA = (SC_ROWS - REM) // 128 - NW * (CH // 128)
TILES = CH // TILE_ROWS
SPAN_A = CH + 128
SPAN_B = CH
SPAN_LAST = CH + REM


def _sc_body(xt_hbm, ae_hbm, out_hbm, xbuf, tailbuf, aexp, outbuf,
             sem0, sem1):
    wid = lax.axis_index("s") * NC + lax.axis_index("c")
    base_l = wid * CH + jnp.minimum(wid, EXTRA) * 128
    base = TC_ROWS + base_l
    zero = jnp.zeros((L,), jnp.float32)

    is_a = wid < EXTRA
    is_last = wid == NW - 1
    tail_row = pl.multiple_of(base + CH, 128)

    pltpu.async_copy(xt_hbm.at[:, pl.ds(pl.multiple_of(base, 128), CH)],
                     xbuf, sem0)
    pltpu.sync_copy(ae_hbm, aexp)

    @pl.when(is_a)
    def _():
        pltpu.async_copy(xt_hbm.at[:, pl.ds(tail_row, 128)], tailbuf, sem1)

    @pl.when(is_last)
    def _():
        for s in range(NSLAB):
            pltpu.async_copy(xt_hbm.at[pl.ds(8 * s, 8), pl.ds(tail_row, REM)],
                             tailbuf.at[pl.ds(8 * s, 8), pl.ds(0, REM)], sem1)
        pltpu.async_copy(xt_hbm.at[pl.ds(8 * NSLAB, 2), pl.ds(tail_row, REM)],
                         tailbuf.at[pl.ds(8 * NSLAB, 2), pl.ds(0, REM)], sem1)

    pltpu.make_async_copy(xt_hbm.at[:, pl.ds(0, CH)], xbuf, sem0).wait()

    @pl.loop(0, TILES)
    def _(t):
        cbase = t * TILE_ROWS
        accs = [zero] * G
        for k in range(E):
            aev = aexp[pl.ds(L * k, L)]
            for j in range(G):
                accs[j] = accs[j] + xbuf[k, pl.ds(cbase + L * j, L)] * aev
        for j in range(G):
            outbuf[pl.ds(cbase + L * j, L)] = accs[j]

    @pl.when(is_a)
    def _():
        pltpu.make_async_copy(xt_hbm.at[:, pl.ds(0, 128)], tailbuf,
                              sem1).wait()

    @pl.when(is_last)
    def _():
        for s in range(NSLAB):
            pltpu.make_async_copy(
                xt_hbm.at[pl.ds(0, 8), pl.ds(0, REM)],
                tailbuf.at[pl.ds(8 * s, 8), pl.ds(0, REM)], sem1).wait()
        pltpu.make_async_copy(
            xt_hbm.at[pl.ds(0, 2), pl.ds(0, REM)],
            tailbuf.at[pl.ds(8 * NSLAB, 2), pl.ds(0, REM)], sem1).wait()

    tail_n = jnp.where(is_a, 128, jnp.where(is_last, REM, 0))

    @pl.loop(0, tail_n // L)
    def _(g):
        acc0 = zero
        acc1 = zero
        for k in range(0, E, 2):
            aev0 = aexp[pl.ds(L * k, L)]
            aev1 = aexp[pl.ds(L * (k + 1), L)]
            acc0 = acc0 + tailbuf[k, pl.ds(g * L, L)] * aev0
            acc1 = acc1 + tailbuf[k + 1, pl.ds(g * L, L)] * aev1
        outbuf[pl.ds(CH + g * L, L)] = acc0 + acc1

    @pl.when(is_a)
    def _():
        pltpu.sync_copy(outbuf.at[pl.ds(0, SPAN_A)],
                        out_hbm.at[pl.ds(base_l, SPAN_A)])

    @pl.when(jnp.logical_and(jnp.logical_not(is_a), jnp.logical_not(is_last)))
    def _():
        pltpu.sync_copy(outbuf.at[pl.ds(0, SPAN_B)],
                        out_hbm.at[pl.ds(base_l, SPAN_B)])

    @pl.when(is_last)
    def _():
        pltpu.sync_copy(outbuf.at[pl.ds(0, SPAN_LAST)],
                        out_hbm.at[pl.ds(base_l, SPAN_LAST)])


@functools.partial(
    pl.kernel,
    out_type=jax.ShapeDtypeStruct((SC_ROWS,), jnp.float32),
    mesh=plsc.VectorSubcoreMesh(core_axis_name="c", subcore_axis_name="s"),
    compiler_params=pltpu.CompilerParams(
        needs_layout_passes=False, use_tc_tiling_on_sc=True),
    scratch_types=[
        pltpu.VMEM((E, CH), jnp.float32),
        pltpu.VMEM((E, 128), jnp.float32),
        pltpu.VMEM((E * L,), jnp.float32),
        pltpu.VMEM((SPAN_A,), jnp.float32),
        pltpu.SemaphoreType.DMA,
        pltpu.SemaphoreType.DMA,
    ],
)
def _sc_matvec(xt, ae_exp, out_flat, xbuf, tailbuf, aexp, outbuf,
               sem0, sem1):
    _sc_body(xt, ae_exp, out_flat, xbuf, tailbuf, aexp, outbuf, sem0, sem1)


def _tc_body(xt_ref, ae_ref, out_ref):
    out_ref[...] = jnp.sum(xt_ref[...] * ae_ref[...], axis=0, keepdims=True)


_tc_matvec = pl.pallas_call(
    _tc_body,
    grid=(TC_GRID,),
    in_specs=[
        pl.BlockSpec((E, BC), lambda i: (0, i)),
        pl.BlockSpec((E, 1), lambda i: (0, 0)),
    ],
    out_specs=pl.BlockSpec((1, BC), lambda i: (0, i)),
    out_shape=jax.ShapeDtypeStruct((1, TC_ROWS), jnp.float32),
)


@jax.jit
def kernel(x, atomic_energies):
    xt = x.T
    ae_exp = jnp.broadcast_to(atomic_energies[:, None], (E, L)).reshape(E * L)
    out_sc = _sc_matvec(xt, ae_exp)
    out_tc = _tc_matvec(xt, atomic_energies[:, None])
    out = jnp.concatenate([out_tc[0], out_sc])
    return out[:, None]
